# Initial kernel scaffold; baseline (speedup 1.0000x reference)
import jax
import jax.numpy as jnp
from jax.experimental import pallas as pl


def kernel(logits, labels):
    raise NotImplementedError("write your pallas kernel here")


# speedup vs baseline: 9.9212x; 9.9212x over previous
import functools

import jax
import jax.numpy as jnp
from jax import lax
from jax.experimental import pallas as pl
from jax.experimental.pallas import tpu as pltpu
from jax.experimental.pallas import tpu_sc as plsc

B = 8
N = 512 * 512
K = 2048
L = 16
TPI = 4
NPT = N // TPI
CH = 8192
NCHUNK = NPT // CH
VPC = CH // L
KL = K * L

_F32_MIN = -3.0e38


def _body(logits_hbm, labels_hbm, out_hbm, lbuf, ybuf, hc, hp, hre, rc, rp, rr,
          stg, pbuf, sh_c, sh_p, sh_r):
    cid = lax.axis_index("c")
    sid = lax.axis_index("s")
    img_local = sid // TPI
    seg = sid % TPI
    img = cid * 4 + img_local
    base = img * N + seg * NPT

    lanes = lax.iota(jnp.int32, L)
    lanes_k = lanes * K
    zero_f = jnp.zeros((L,), jnp.float32)
    zero_i = jnp.zeros((L,), jnp.int32)

    def zb(i, _):
        hc[pl.ds(i * L, L)] = zero_f
        hp[pl.ds(i * L, L)] = zero_f
        hre[pl.ds(i * L, L)] = zero_f
        return 0
    lax.fori_loop(0, KL // L, zb, 0)

    maxv = jnp.full((L,), _F32_MIN, jnp.float32)
    sumv = zero_f
    img_base = img * N
    for ch in range(TPI * NCHUNK):
        off = img_base + ch * CH
        pltpu.sync_copy(logits_hbm.at[pl.ds(off, CH)], lbuf)
        pltpu.sync_copy(labels_hbm.at[pl.ds(off, CH)], ybuf)

        def abody(i, carry):
            mx, sm = carry
            lg = lbuf[pl.ds(i * L, L)]
            yf = ybuf[pl.ds(i * L, L)].astype(jnp.float32)
            e = 1.0 - lg * (2.0 * yf - 1.0)
            return jnp.maximum(mx, e), sm + yf
        maxv, sumv = lax.fori_loop(0, VPC, abody, (maxv, sumv))

    m_vec = jnp.maximum(jnp.full((L,), jnp.max(maxv)), 1e-30)
    p_vec = jnp.full((L,), jnp.sum(sumv))
    scale = jnp.float32(K) / m_vec

    for ch in range(NCHUNK):
        off = base + ch * CH
        pltpu.sync_copy(logits_hbm.at[pl.ds(off, CH)], lbuf)
        pltpu.sync_copy(labels_hbm.at[pl.ds(off, CH)], ybuf)

        def bbody(i, _):
            lg = lbuf[pl.ds(i * L, L)]
            y = ybuf[pl.ds(i * L, L)]
            yf = y.astype(jnp.float32)
            e = 1.0 - lg * (2.0 * yf - 1.0)
            mask = e > 0.0
            relu = jnp.where(mask, e, 0.0)
            binf = jnp.clip(e * scale, 0.0, jnp.float32(K - 1))
            bi = binf.astype(jnp.int32)
            addr = lanes_k + (jnp.int32(K - 1) - bi)
            vc = jnp.where(mask, 1.0, 0.0)
            vp = jnp.where(mask, yf, 0.0)
            plsc.addupdate_scatter(hc, [addr], vc)
            plsc.addupdate_scatter(hp, [addr], vp)
            plsc.addupdate_scatter(hre, [addr], relu)
            return 0
        lax.fori_loop(0, VPC, bbody, 0)

    def rbody(g, _):
        sl = pl.ds(g * L, L)
        cacc = zero_f
        pacc = zero_f
        racc = zero_f
        for ln in range(L):
            cacc = cacc + hc[pl.ds(ln * K + g * L, L)]
            pacc = pacc + hp[pl.ds(ln * K + g * L, L)]
            racc = racc + hre[pl.ds(ln * K + g * L, L)]
        rc[sl] = cacc
        rp[sl] = pacc
        rr[sl] = racc
        return 0
    lax.fori_loop(0, K // L, rbody, 0)

    pltpu.sync_copy(rc, sh_c.at[img_local, seg])
    pltpu.sync_copy(rp, sh_p.at[img_local, seg])
    pltpu.sync_copy(rr, sh_r.at[img_local, seg])
    plsc.subcore_barrier()
    plsc.subcore_barrier()

    @pl.when(seg == 0)
    def _():
        for j in range(1, TPI):
            pltpu.sync_copy(sh_c.at[img_local, j], stg)

            def mc_(i, _):
                sl = pl.ds(i * L, L)
                rc[sl] = rc[sl] + stg[sl]
                return 0
            lax.fori_loop(0, K // L, mc_, 0)
            pltpu.sync_copy(sh_p.at[img_local, j], stg)

            def mp_(i, _):
                sl = pl.ds(i * L, L)
                rp[sl] = rp[sl] + stg[sl]
                return 0
            lax.fori_loop(0, K // L, mp_, 0)
            pltpu.sync_copy(sh_r.at[img_local, j], stg)

            def mr_(i, _):
                sl = pl.ds(i * L, L)
                rr[sl] = rr[sl] + stg[sl]
                return 0
            lax.fori_loop(0, K // L, mr_, 0)

        def jacc(a, c):
            return jnp.where(a > 0.0,
                             1.0 - (p_vec - c) / jnp.maximum(p_vec + a - c, 1.0),
                             0.0)

        def sbody(g, carry):
            ca, cc, acc, accr = carry
            sl = pl.ds(g * L, L)
            cnt = rc[sl]
            pos = rp[sl]
            rel = rr[sl]
            a_in = ca + plsc.cumsum(cnt)
            c_in = cc + plsc.cumsum(pos)
            a_ex = a_in - cnt
            c_ex = c_in - pos
            term = jnp.where(cnt > 0.0,
                             rel * (jacc(a_in, c_in) - jacc(a_ex, c_ex))
                             / jnp.maximum(cnt, 1.0),
                             0.0)
            ca = ca + jnp.full((L,), jnp.sum(cnt))
            cc = cc + jnp.full((L,), jnp.sum(pos))
            return ca, cc, acc + term, accr + rel
        ca_f, cc_f, lossacc, accr_f = lax.fori_loop(
            0, K // L, sbody, (zero_f, zero_f, zero_f, zero_f))

        iot = lanes
        dbg = jnp.full((L,), jnp.sum(lossacc))
        dbg = jnp.where(iot == 1, p_vec, dbg)
        dbg = jnp.where(iot == 2, m_vec, dbg)
        dbg = jnp.where(iot == 3, ca_f, dbg)
        dbg = jnp.where(iot == 4, cc_f, dbg)
        dbg = jnp.where(iot == 5, jnp.full((L,), jnp.sum(accr_f)), dbg)
        pbuf[...] = dbg
        pltpu.sync_copy(pbuf, out_hbm.at[img])


@jax.jit
def _lovasz_sc(logits_flat, labels_flat):
    k = functools.partial(
        pl.kernel,
        out_type=jax.ShapeDtypeStruct((B, L), jnp.float32),
        mesh=plsc.VectorSubcoreMesh(core_axis_name="c", subcore_axis_name="s"),
        compiler_params=pltpu.CompilerParams(needs_layout_passes=False),
        scratch_types=dict(
            lbuf=pltpu.VMEM((CH,), jnp.float32),
            ybuf=pltpu.VMEM((CH,), jnp.int32),
            hc=pltpu.VMEM((KL,), jnp.float32),
            hp=pltpu.VMEM((KL,), jnp.float32),
            hre=pltpu.VMEM((KL,), jnp.float32),
            rc=pltpu.VMEM((K,), jnp.float32),
            rp=pltpu.VMEM((K,), jnp.float32),
            rr=pltpu.VMEM((K,), jnp.float32),
            stg=pltpu.VMEM((K,), jnp.float32),
            pbuf=pltpu.VMEM((L,), jnp.float32),
            sh_c=pltpu.VMEM_SHARED((4, TPI, K), jnp.float32),
            sh_p=pltpu.VMEM_SHARED((4, TPI, K), jnp.float32),
            sh_r=pltpu.VMEM_SHARED((4, TPI, K), jnp.float32),
        ),
    )(_body)
    return k(logits_flat, labels_flat)


def kernel(logits, labels):
    logits_flat = logits.reshape(-1)
    labels_flat = labels.reshape(-1).astype(jnp.int32)
    per_image = _lovasz_sc(logits_flat, labels_flat)
    return jnp.mean(per_image[:, 0])


# speedup vs baseline: 16.5107x; 1.6642x over previous
import functools

import jax
import jax.numpy as jnp
from jax import lax
from jax.experimental import pallas as pl
from jax.experimental.pallas import tpu as pltpu
from jax.experimental.pallas import tpu_sc as plsc

B = 8
N = 512 * 512
K = 2048
L = 16
TPI = 4
NPT = N // TPI
CH = 8192
NCHUNK = NPT // CH
VPC = CH // L
KL = K * L

_F32_MIN = -3.0e38


def _tc_stats_body(lg_ref, lb_ref, mx_ref, p_ref):
    lg = lg_ref[0]
    y = lb_ref[0].astype(jnp.float32)
    e = 1.0 - lg * (2.0 * y - 1.0)
    mx_ref[0] = jnp.full((8, 128), jnp.max(e), jnp.float32)
    p_ref[0] = jnp.full((8, 128), jnp.sum(y), jnp.float32)


@jax.jit
def _tc_stats(lg3, lb3):
    return pl.pallas_call(
        _tc_stats_body,
        grid=(B,),
        in_specs=[
            pl.BlockSpec((1, N // 128, 128), lambda i: (i, 0, 0)),
            pl.BlockSpec((1, N // 128, 128), lambda i: (i, 0, 0)),
        ],
        out_specs=[
            pl.BlockSpec((1, 8, 128), lambda i: (i, 0, 0)),
            pl.BlockSpec((1, 8, 128), lambda i: (i, 0, 0)),
        ],
        out_shape=[
            jax.ShapeDtypeStruct((B, 8, 128), jnp.float32),
            jax.ShapeDtypeStruct((B, 8, 128), jnp.float32),
        ],
    )(lg3, lb3)


def _body(logits_hbm, labels_hbm, maxs_hbm, ps_hbm, out_hbm, lbuf, ybuf,
          hcp, hre, rc, rp, rr, stg, pbuf, mbuf, m2buf, sh_c, sh_p, sh_r):
    cid = lax.axis_index("c")
    sid = lax.axis_index("s")
    img_local = sid // TPI
    seg = sid % TPI
    img = cid * 4 + img_local
    base = img * N + seg * NPT

    lanes = lax.iota(jnp.int32, L)
    lanes_k = lanes * K
    zero_f = jnp.zeros((L,), jnp.float32)
    zero_i = jnp.zeros((L,), jnp.int32)

    def zb(i, _):
        hcp[pl.ds(i * L, L)] = zero_i
        hre[pl.ds(i * L, L)] = zero_f
        return 0
    lax.fori_loop(0, KL // L, zb, 0)

    pltpu.sync_copy(maxs_hbm.at[img, 0, pl.ds(0, L)], mbuf)
    pltpu.sync_copy(ps_hbm.at[img, 0, pl.ds(0, L)], m2buf)
    m_vec = jnp.maximum(mbuf[...], 1e-30)
    p_vec = m2buf[...]
    scale = jnp.float32(K) / m_vec

    for ch in range(NCHUNK):
        off = base + ch * CH
        pltpu.sync_copy(logits_hbm.at[pl.ds(off, CH)], lbuf)
        pltpu.sync_copy(labels_hbm.at[pl.ds(off, CH)], ybuf)

        def bbody(i, _):
            lg = lbuf[pl.ds(i * L, L)]
            y = ybuf[pl.ds(i * L, L)]
            yf = y.astype(jnp.float32)
            e = 1.0 - lg * (2.0 * yf - 1.0)
            mask = e > 0.0
            relu = jnp.where(mask, e, 0.0)
            binf = jnp.clip(e * scale, 0.0, jnp.float32(K - 1))
            bi = binf.astype(jnp.int32)
            addr = lanes_k + (jnp.int32(K - 1) - bi)
            vcp = jnp.where(mask, 1 + (y << 13), 0)
            plsc.addupdate_scatter(hcp, [addr], vcp)
            plsc.addupdate_scatter(hre, [addr], relu)
            return 0
        lax.fori_loop(0, VPC, bbody, 0)

    def rbody(g, _):
        sl = pl.ds(g * L, L)
        cacc = zero_f
        pacc = zero_f
        racc = zero_f
        for ln in range(L):
            v = hcp[pl.ds(ln * K + g * L, L)]
            cacc = cacc + (v & 8191).astype(jnp.float32)
            pacc = pacc + (v >> 13).astype(jnp.float32)
            racc = racc + hre[pl.ds(ln * K + g * L, L)]
        rc[sl] = cacc
        rp[sl] = pacc
        rr[sl] = racc
        return 0
    lax.fori_loop(0, K // L, rbody, 0)

    pltpu.sync_copy(rc, sh_c.at[img_local, seg])
    pltpu.sync_copy(rp, sh_p.at[img_local, seg])
    pltpu.sync_copy(rr, sh_r.at[img_local, seg])
    plsc.subcore_barrier()
    plsc.subcore_barrier()

    @pl.when(seg == 0)
    def _():
        for j in range(1, TPI):
            pltpu.sync_copy(sh_c.at[img_local, j], stg)

            def mc_(i, _):
                sl = pl.ds(i * L, L)
                rc[sl] = rc[sl] + stg[sl]
                return 0
            lax.fori_loop(0, K // L, mc_, 0)
            pltpu.sync_copy(sh_p.at[img_local, j], stg)

            def mp_(i, _):
                sl = pl.ds(i * L, L)
                rp[sl] = rp[sl] + stg[sl]
                return 0
            lax.fori_loop(0, K // L, mp_, 0)
            pltpu.sync_copy(sh_r.at[img_local, j], stg)

            def mr_(i, _):
                sl = pl.ds(i * L, L)
                rr[sl] = rr[sl] + stg[sl]
                return 0
            lax.fori_loop(0, K // L, mr_, 0)

        def jacc(a, c):
            return jnp.where(a > 0.0,
                             1.0 - (p_vec - c) / jnp.maximum(p_vec + a - c, 1.0),
                             0.0)

        def sbody(g, carry):
            ca, cc, acc, accr = carry
            sl = pl.ds(g * L, L)
            cnt = rc[sl]
            pos = rp[sl]
            rel = rr[sl]
            a_in = ca + plsc.cumsum(cnt)
            c_in = cc + plsc.cumsum(pos)
            a_ex = a_in - cnt
            c_ex = c_in - pos
            term = jnp.where(cnt > 0.0,
                             rel * (jacc(a_in, c_in) - jacc(a_ex, c_ex))
                             / jnp.maximum(cnt, 1.0),
                             0.0)
            ca = ca + jnp.full((L,), jnp.sum(cnt))
            cc = cc + jnp.full((L,), jnp.sum(pos))
            return ca, cc, acc + term, accr + rel
        ca_f, cc_f, lossacc, accr_f = lax.fori_loop(
            0, K // L, sbody, (zero_f, zero_f, zero_f, zero_f))

        iot = lanes
        dbg = jnp.full((L,), jnp.sum(lossacc))
        dbg = jnp.where(iot == 1, p_vec, dbg)
        dbg = jnp.where(iot == 2, m_vec, dbg)
        dbg = jnp.where(iot == 3, ca_f, dbg)
        dbg = jnp.where(iot == 4, cc_f, dbg)
        dbg = jnp.where(iot == 5, jnp.full((L,), jnp.sum(accr_f)), dbg)
        pbuf[...] = dbg
        pltpu.sync_copy(pbuf, out_hbm.at[img])


@jax.jit
def _lovasz_sc(logits_flat, labels_flat, maxs, ps):
    k = functools.partial(
        pl.kernel,
        out_type=jax.ShapeDtypeStruct((B, L), jnp.float32),
        mesh=plsc.VectorSubcoreMesh(core_axis_name="c", subcore_axis_name="s"),
        compiler_params=pltpu.CompilerParams(needs_layout_passes=False),
        scratch_types=dict(
            lbuf=pltpu.VMEM((CH,), jnp.float32),
            ybuf=pltpu.VMEM((CH,), jnp.int32),
            hcp=pltpu.VMEM((KL,), jnp.int32),
            hre=pltpu.VMEM((KL,), jnp.float32),
            rc=pltpu.VMEM((K,), jnp.float32),
            rp=pltpu.VMEM((K,), jnp.float32),
            rr=pltpu.VMEM((K,), jnp.float32),
            stg=pltpu.VMEM((K,), jnp.float32),
            pbuf=pltpu.VMEM((L,), jnp.float32),
            mbuf=pltpu.VMEM((L,), jnp.float32),
            m2buf=pltpu.VMEM((L,), jnp.float32),
            sh_c=pltpu.VMEM_SHARED((4, TPI, K), jnp.float32),
            sh_p=pltpu.VMEM_SHARED((4, TPI, K), jnp.float32),
            sh_r=pltpu.VMEM_SHARED((4, TPI, K), jnp.float32),
        ),
    )(_body)
    return k(logits_flat, labels_flat, maxs, ps)


def kernel(logits, labels):
    labels_i = labels.astype(jnp.int32)
    maxs, ps = _tc_stats(logits.reshape(B, N // 128, 128),
                         labels_i.reshape(B, N // 128, 128))
    per_image = _lovasz_sc(logits.reshape(-1), labels_i.reshape(-1), maxs, ps)
    return jnp.mean(per_image[:, 0])


# speedup vs baseline: 18.3085x; 1.1089x over previous
import functools

import jax
import jax.numpy as jnp
from jax import lax
from jax.experimental import pallas as pl
from jax.experimental.pallas import tpu as pltpu
from jax.experimental.pallas import tpu_sc as plsc

B = 8
N = 512 * 512
K = 2048
L = 16
TPI = 4
NPT = N // TPI
CH = 8192
NCHUNK = NPT // CH
VPC = CH // L
KL = K * L

_F32_MIN = -3.0e38


def _tc_stats_body(lg_ref, lb_ref, mx_ref, p_ref):
    lg = lg_ref[0]
    y = lb_ref[0].astype(jnp.float32)
    e = 1.0 - lg * (2.0 * y - 1.0)
    mx_ref[0] = jnp.full((8, 128), jnp.max(e), jnp.float32)
    p_ref[0] = jnp.full((8, 128), jnp.sum(y), jnp.float32)


@jax.jit
def _tc_stats(lg3, lb3):
    return pl.pallas_call(
        _tc_stats_body,
        grid=(B,),
        in_specs=[
            pl.BlockSpec((1, N // 128, 128), lambda i: (i, 0, 0)),
            pl.BlockSpec((1, N // 128, 128), lambda i: (i, 0, 0)),
        ],
        out_specs=[
            pl.BlockSpec((1, 8, 128), lambda i: (i, 0, 0)),
            pl.BlockSpec((1, 8, 128), lambda i: (i, 0, 0)),
        ],
        out_shape=[
            jax.ShapeDtypeStruct((B, 8, 128), jnp.float32),
            jax.ShapeDtypeStruct((B, 8, 128), jnp.float32),
        ],
    )(lg3, lb3)


def _body(logits_hbm, labels_hbm, maxs_hbm, ps_hbm, out_hbm, lbuf, ybuf,
          hcp, hre, rc, rp, rr, stg, pbuf, mbuf, m2buf, sem0, sem1,
          sh_c, sh_p, sh_r):
    cid = lax.axis_index("c")
    sid = lax.axis_index("s")
    img_local = sid // TPI
    seg = sid % TPI
    img = cid * 4 + img_local
    base = img * N + seg * NPT

    lanes = lax.iota(jnp.int32, L)
    lanes_k = lanes * K
    zero_f = jnp.zeros((L,), jnp.float32)
    zero_i = jnp.zeros((L,), jnp.int32)

    def zb(i, _):
        for u in range(8):
            sl = pl.ds(i * (8 * L) + u * L, L)
            hcp[sl] = zero_i
            hre[sl] = zero_f
        return 0
    lax.fori_loop(0, KL // (8 * L), zb, 0)

    pltpu.sync_copy(maxs_hbm.at[img, 0, pl.ds(0, L)], mbuf)
    pltpu.sync_copy(ps_hbm.at[img, 0, pl.ds(0, L)], m2buf)
    m_vec = jnp.maximum(mbuf[...], 1e-30)
    p_vec = m2buf[...]
    scale = jnp.float32(K) / m_vec

    lk1 = lanes_k + jnp.int32(K - 1)
    sems = (sem0, sem1)

    def start(ch, slot):
        off = base + ch * CH
        c0 = pltpu.make_async_copy(logits_hbm.at[pl.ds(off, CH)],
                                   lbuf.at[slot], sems[slot])
        c1 = pltpu.make_async_copy(labels_hbm.at[pl.ds(off, CH)],
                                   ybuf.at[slot], sems[slot])
        c0.start()
        c1.start()
        return c0, c1

    def bstep(slot, i, u):
        sl = pl.ds((2 * i + u) * L, L)
        lg = lbuf[slot, sl]
        y = ybuf[slot, sl]
        yf = y.astype(jnp.float32)
        e = 1.0 - lg * (2.0 * yf - 1.0)
        mask = e > 0.0
        relu = jnp.where(mask, e, 0.0)
        binf = jnp.clip(e * scale, 0.0, jnp.float32(K - 1))
        bi = binf.astype(jnp.int32)
        addr = lk1 - bi
        vcp = jnp.where(mask, 1 + (y << 13), 0)
        plsc.addupdate_scatter(hcp, [addr], vcp)
        plsc.addupdate_scatter(hre, [addr], relu)

    pend = start(0, 0)
    for ch in range(NCHUNK):
        slot = ch % 2
        pend[0].wait()
        pend[1].wait()
        if ch + 1 < NCHUNK:
            pend = start(ch + 1, 1 - slot)

        def bbody(i, _, slot=slot):
            bstep(slot, i, 0)
            bstep(slot, i, 1)
            return 0
        lax.fori_loop(0, VPC // 2, bbody, 0)

    def rbody(g, _):
        sl = pl.ds(g * L, L)
        cacc = zero_f
        pacc = zero_f
        racc = zero_f
        for ln in range(L):
            v = hcp[pl.ds(ln * K + g * L, L)]
            cacc = cacc + (v & 8191).astype(jnp.float32)
            pacc = pacc + (v >> 13).astype(jnp.float32)
            racc = racc + hre[pl.ds(ln * K + g * L, L)]
        rc[sl] = cacc
        rp[sl] = pacc
        rr[sl] = racc
        return 0
    lax.fori_loop(0, K // L, rbody, 0)

    pltpu.sync_copy(rc, sh_c.at[img_local, seg])
    pltpu.sync_copy(rp, sh_p.at[img_local, seg])
    pltpu.sync_copy(rr, sh_r.at[img_local, seg])
    plsc.subcore_barrier()
    plsc.subcore_barrier()

    @pl.when(seg == 0)
    def _():
        for j in range(1, TPI):
            pltpu.sync_copy(sh_c.at[img_local, j], stg)

            def mc_(i, _):
                sl = pl.ds(i * L, L)
                rc[sl] = rc[sl] + stg[sl]
                return 0
            lax.fori_loop(0, K // L, mc_, 0)
            pltpu.sync_copy(sh_p.at[img_local, j], stg)

            def mp_(i, _):
                sl = pl.ds(i * L, L)
                rp[sl] = rp[sl] + stg[sl]
                return 0
            lax.fori_loop(0, K // L, mp_, 0)
            pltpu.sync_copy(sh_r.at[img_local, j], stg)

            def mr_(i, _):
                sl = pl.ds(i * L, L)
                rr[sl] = rr[sl] + stg[sl]
                return 0
            lax.fori_loop(0, K // L, mr_, 0)

        def jacc(a, c):
            return jnp.where(a > 0.0,
                             1.0 - (p_vec - c) / jnp.maximum(p_vec + a - c, 1.0),
                             0.0)

        def sbody(g, carry):
            ca, cc, acc, accr = carry
            sl = pl.ds(g * L, L)
            cnt = rc[sl]
            pos = rp[sl]
            rel = rr[sl]
            a_in = ca + plsc.cumsum(cnt)
            c_in = cc + plsc.cumsum(pos)
            a_ex = a_in - cnt
            c_ex = c_in - pos
            term = jnp.where(cnt > 0.0,
                             rel * (jacc(a_in, c_in) - jacc(a_ex, c_ex))
                             / jnp.maximum(cnt, 1.0),
                             0.0)
            ca = ca + jnp.full((L,), jnp.sum(cnt))
            cc = cc + jnp.full((L,), jnp.sum(pos))
            return ca, cc, acc + term, accr + rel
        ca_f, cc_f, lossacc, accr_f = lax.fori_loop(
            0, K // L, sbody, (zero_f, zero_f, zero_f, zero_f))

        iot = lanes
        dbg = jnp.full((L,), jnp.sum(lossacc))
        dbg = jnp.where(iot == 1, p_vec, dbg)
        dbg = jnp.where(iot == 2, m_vec, dbg)
        dbg = jnp.where(iot == 3, ca_f, dbg)
        dbg = jnp.where(iot == 4, cc_f, dbg)
        dbg = jnp.where(iot == 5, jnp.full((L,), jnp.sum(accr_f)), dbg)
        pbuf[...] = dbg
        pltpu.sync_copy(pbuf, out_hbm.at[img])


@jax.jit
def _lovasz_sc(logits_flat, labels_flat, maxs, ps):
    k = functools.partial(
        pl.kernel,
        out_type=jax.ShapeDtypeStruct((B, L), jnp.float32),
        mesh=plsc.VectorSubcoreMesh(core_axis_name="c", subcore_axis_name="s"),
        compiler_params=pltpu.CompilerParams(needs_layout_passes=False),
        scratch_types=dict(
            lbuf=pltpu.VMEM((2, CH), jnp.float32),
            ybuf=pltpu.VMEM((2, CH), jnp.int32),
            hcp=pltpu.VMEM((KL,), jnp.int32),
            hre=pltpu.VMEM((KL,), jnp.float32),
            rc=pltpu.VMEM((K,), jnp.float32),
            rp=pltpu.VMEM((K,), jnp.float32),
            rr=pltpu.VMEM((K,), jnp.float32),
            stg=pltpu.VMEM((K,), jnp.float32),
            pbuf=pltpu.VMEM((L,), jnp.float32),
            mbuf=pltpu.VMEM((L,), jnp.float32),
            m2buf=pltpu.VMEM((L,), jnp.float32),
            sem0=pltpu.SemaphoreType.DMA,
            sem1=pltpu.SemaphoreType.DMA,
            sh_c=pltpu.VMEM_SHARED((4, TPI, K), jnp.float32),
            sh_p=pltpu.VMEM_SHARED((4, TPI, K), jnp.float32),
            sh_r=pltpu.VMEM_SHARED((4, TPI, K), jnp.float32),
        ),
    )(_body)
    return k(logits_flat, labels_flat, maxs, ps)


def kernel(logits, labels):
    labels_i = labels.astype(jnp.int32)
    maxs, ps = _tc_stats(logits.reshape(B, N // 128, 128),
                         labels_i.reshape(B, N // 128, 128))
    per_image = _lovasz_sc(logits.reshape(-1), labels_i.reshape(-1), maxs, ps)
    return jnp.mean(per_image[:, 0])


# speedup vs baseline: 29.9677x; 1.6368x over previous
import functools

import jax
import jax.numpy as jnp
from jax import lax
from jax.experimental import pallas as pl
from jax.experimental.pallas import tpu as pltpu
from jax.experimental.pallas import tpu_sc as plsc

B = 8
N = 512 * 512
K = 2048
L = 16
TPI = 4
NPT = N // TPI
CH = 8192
NCHUNK = NPT // CH
VPC = CH // L
KL = K * L

_F32_MIN = -3.0e38


def _tc_stats_body(lg_ref, lb_ref, mx_ref, p_ref):
    lg = lg_ref[0]
    y = lb_ref[0].astype(jnp.float32)
    e = 1.0 - lg * (2.0 * y - 1.0)
    mx_ref[0] = jnp.full((8, 128), jnp.max(e), jnp.float32)
    p_ref[0] = jnp.full((8, 128), jnp.sum(y), jnp.float32)


@jax.jit
def _tc_stats(lg3, lb3):
    return pl.pallas_call(
        _tc_stats_body,
        grid=(B,),
        in_specs=[
            pl.BlockSpec((1, N // 128, 128), lambda i: (i, 0, 0)),
            pl.BlockSpec((1, N // 128, 128), lambda i: (i, 0, 0)),
        ],
        out_specs=[
            pl.BlockSpec((1, 8, 128), lambda i: (i, 0, 0)),
            pl.BlockSpec((1, 8, 128), lambda i: (i, 0, 0)),
        ],
        out_shape=[
            jax.ShapeDtypeStruct((B, 8, 128), jnp.float32),
            jax.ShapeDtypeStruct((B, 8, 128), jnp.float32),
        ],
    )(lg3, lb3)


def _body(logits_hbm, labels_hbm, maxs_hbm, ps_hbm, out_hbm, lbuf, ybuf,
          hcp, hre, rc, rp, rr, stg, pbuf, mbuf, m2buf, sem0, sem1,
          sh_c, sh_p, sh_r):
    cid = lax.axis_index("c")
    sid = lax.axis_index("s")
    img_local = sid // TPI
    seg = sid % TPI
    img = cid * 4 + img_local
    base = img * N + seg * NPT

    lanes = lax.iota(jnp.int32, L)
    lanes_k = lanes * K
    zero_f = jnp.zeros((L,), jnp.float32)
    zero_i = jnp.zeros((L,), jnp.int32)

    def zb(i, _):
        for u in range(8):
            sl = pl.ds(i * (8 * L) + u * L, L)
            hcp[sl] = zero_i
            hre[sl] = zero_f
        return 0
    lax.fori_loop(0, KL // (8 * L), zb, 0)

    pltpu.sync_copy(maxs_hbm.at[img, 0, pl.ds(0, L)], mbuf)
    pltpu.sync_copy(ps_hbm.at[img, 0, pl.ds(0, L)], m2buf)
    m_vec = jnp.maximum(mbuf[...], 1e-30)
    p_vec = m2buf[...]
    scale = jnp.float32(K) / m_vec

    lk1 = lanes_k + jnp.int32(K - 1)
    sems = (sem0, sem1)

    def start(ch, slot):
        off = base + ch * CH
        c0 = pltpu.make_async_copy(logits_hbm.at[pl.ds(off, CH)],
                                   lbuf.at[slot], sems[slot])
        c1 = pltpu.make_async_copy(labels_hbm.at[pl.ds(off, CH)],
                                   ybuf.at[slot], sems[slot])
        c0.start()
        c1.start()
        return c0, c1

    def bstep(slot, i):
        sl = pl.ds(i * L, L)
        lg = lbuf[slot, sl]
        y = ybuf[slot, sl]
        yf = y.astype(jnp.float32)
        e = 1.0 - lg * (2.0 * yf - 1.0)
        mask = e > 0.0
        relu = jnp.where(mask, e, 0.0)
        binf = jnp.clip(e * scale, 0.0, jnp.float32(K - 1))
        bi = binf.astype(jnp.int32)
        addr = lk1 - bi
        vcp = jnp.where(mask, 1 + (y << 13), 0)
        plsc.addupdate_scatter(hcp, [addr], vcp)
        plsc.addupdate_scatter(hre, [addr], relu)

    pend = start(0, 0)
    for ch in range(NCHUNK):
        slot = ch % 2
        pend[0].wait()
        pend[1].wait()
        if ch + 1 < NCHUNK:
            pend = start(ch + 1, 1 - slot)

        @plsc.parallel_loop(0, VPC, unroll=4)
        def _(i, slot=slot):
            bstep(slot, i)

    def rbody(g, _):
        sl = pl.ds(g * L, L)
        cacc = zero_f
        pacc = zero_f
        racc = zero_f
        for ln in range(L):
            v = hcp[pl.ds(ln * K + g * L, L)]
            cacc = cacc + (v & 8191).astype(jnp.float32)
            pacc = pacc + (v >> 13).astype(jnp.float32)
            racc = racc + hre[pl.ds(ln * K + g * L, L)]
        rc[sl] = cacc
        rp[sl] = pacc
        rr[sl] = racc
        return 0
    lax.fori_loop(0, K // L, rbody, 0)

    pltpu.sync_copy(rc, sh_c.at[img_local, seg])
    pltpu.sync_copy(rp, sh_p.at[img_local, seg])
    pltpu.sync_copy(rr, sh_r.at[img_local, seg])
    plsc.subcore_barrier()
    plsc.subcore_barrier()

    @pl.when(seg == 0)
    def _():
        for j in range(1, TPI):
            pltpu.sync_copy(sh_c.at[img_local, j], stg)

            def mc_(i, _):
                sl = pl.ds(i * L, L)
                rc[sl] = rc[sl] + stg[sl]
                return 0
            lax.fori_loop(0, K // L, mc_, 0)
            pltpu.sync_copy(sh_p.at[img_local, j], stg)

            def mp_(i, _):
                sl = pl.ds(i * L, L)
                rp[sl] = rp[sl] + stg[sl]
                return 0
            lax.fori_loop(0, K // L, mp_, 0)
            pltpu.sync_copy(sh_r.at[img_local, j], stg)

            def mr_(i, _):
                sl = pl.ds(i * L, L)
                rr[sl] = rr[sl] + stg[sl]
                return 0
            lax.fori_loop(0, K // L, mr_, 0)

        def jacc(a, c):
            return jnp.where(a > 0.0,
                             1.0 - (p_vec - c) / jnp.maximum(p_vec + a - c, 1.0),
                             0.0)

        def sbody(g, carry):
            ca, cc, acc, accr = carry
            sl = pl.ds(g * L, L)
            cnt = rc[sl]
            pos = rp[sl]
            rel = rr[sl]
            a_in = ca + plsc.cumsum(cnt)
            c_in = cc + plsc.cumsum(pos)
            a_ex = a_in - cnt
            c_ex = c_in - pos
            term = jnp.where(cnt > 0.0,
                             rel * (jacc(a_in, c_in) - jacc(a_ex, c_ex))
                             / jnp.maximum(cnt, 1.0),
                             0.0)
            ca = ca + jnp.full((L,), jnp.sum(cnt))
            cc = cc + jnp.full((L,), jnp.sum(pos))
            return ca, cc, acc + term, accr + rel
        ca_f, cc_f, lossacc, accr_f = lax.fori_loop(
            0, K // L, sbody, (zero_f, zero_f, zero_f, zero_f))

        iot = lanes
        dbg = jnp.full((L,), jnp.sum(lossacc))
        dbg = jnp.where(iot == 1, p_vec, dbg)
        dbg = jnp.where(iot == 2, m_vec, dbg)
        dbg = jnp.where(iot == 3, ca_f, dbg)
        dbg = jnp.where(iot == 4, cc_f, dbg)
        dbg = jnp.where(iot == 5, jnp.full((L,), jnp.sum(accr_f)), dbg)
        pbuf[...] = dbg
        pltpu.sync_copy(pbuf, out_hbm.at[img])


@jax.jit
def _lovasz_sc(logits_flat, labels_flat, maxs, ps):
    k = functools.partial(
        pl.kernel,
        out_type=jax.ShapeDtypeStruct((B, L), jnp.float32),
        mesh=plsc.VectorSubcoreMesh(core_axis_name="c", subcore_axis_name="s"),
        compiler_params=pltpu.CompilerParams(needs_layout_passes=False),
        scratch_types=dict(
            lbuf=pltpu.VMEM((2, CH), jnp.float32),
            ybuf=pltpu.VMEM((2, CH), jnp.int32),
            hcp=pltpu.VMEM((KL,), jnp.int32),
            hre=pltpu.VMEM((KL,), jnp.float32),
            rc=pltpu.VMEM((K,), jnp.float32),
            rp=pltpu.VMEM((K,), jnp.float32),
            rr=pltpu.VMEM((K,), jnp.float32),
            stg=pltpu.VMEM((K,), jnp.float32),
            pbuf=pltpu.VMEM((L,), jnp.float32),
            mbuf=pltpu.VMEM((L,), jnp.float32),
            m2buf=pltpu.VMEM((L,), jnp.float32),
            sem0=pltpu.SemaphoreType.DMA,
            sem1=pltpu.SemaphoreType.DMA,
            sh_c=pltpu.VMEM_SHARED((4, TPI, K), jnp.float32),
            sh_p=pltpu.VMEM_SHARED((4, TPI, K), jnp.float32),
            sh_r=pltpu.VMEM_SHARED((4, TPI, K), jnp.float32),
        ),
    )(_body)
    return k(logits_flat, labels_flat, maxs, ps)


def kernel(logits, labels):
    labels_i = labels.astype(jnp.int32)
    maxs, ps = _tc_stats(logits.reshape(B, N // 128, 128),
                         labels_i.reshape(B, N // 128, 128))
    per_image = _lovasz_sc(logits.reshape(-1), labels_i.reshape(-1), maxs, ps)
    return jnp.mean(per_image[:, 0])
